# baseline (speedup 1.0000x reference)
import jax
import jax.numpy as jnp
from jax import lax
from jax.experimental import pallas as pl
from jax.experimental.pallas import tpu as pltpu

_BN = 32000
_NB = 5


def _body(u_hbm, i_hbm, o_hbm, u_v, i_v, o_v, sem_u, sem_i, sem_o):
    n = u_hbm.shape[1]
    steps = n // _BN

    def issue_in(si, b):
        start = si * _BN
        pltpu.async_copy(u_hbm.at[:, pl.ds(start, _BN)], u_v.at[b],
                         sem_u.at[b])
        pltpu.async_copy(i_hbm.at[:, pl.ds(start, _BN)], i_v.at[b],
                         sem_i.at[b])

    def wait_in(b):
        pltpu.make_async_copy(u_hbm.at[:, pl.ds(0, _BN)], u_v.at[b],
                              sem_u.at[b]).wait()
        pltpu.make_async_copy(i_hbm.at[:, pl.ds(0, _BN)], i_v.at[b],
                              sem_i.at[b]).wait()

    def issue_out(si, b):
        pltpu.async_copy(o_v.at[b, 0], o_hbm.at[pl.ds(si * _BN, _BN)],
                         sem_o.at[b])

    def wait_out(b):
        pltpu.make_async_copy(o_v.at[b, 0], o_hbm.at[pl.ds(0, _BN)],
                              sem_o.at[b]).wait()

    for b in range(_NB - 1):
        issue_in(b, b)

    def grp(g, carry):
        for j in range(_NB):
            si = g * _NB + j
            nxt = si + _NB - 1

            @pl.when(nxt < steps)
            def _():
                issue_in(nxt, (j + _NB - 1) % _NB)

            wait_in(j)

            @pl.when(si >= _NB)
            def _():
                wait_out(j)

            o_v[j, 0, :] = jnp.sum(u_v[j] * i_v[j], axis=0)
            issue_out(si, j)
        return carry

    lax.fori_loop(0, steps // _NB, grp, 0)
    for b in range(_NB):
        wait_out(b)


def kernel(gu, gi):
    gu = jnp.squeeze(gu)
    gi = jnp.squeeze(gi)
    n, k = gu.shape
    ut = gu.T
    it = gi.T
    return pl.pallas_call(
        _body,
        in_specs=[
            pl.BlockSpec(memory_space=pltpu.HBM),
            pl.BlockSpec(memory_space=pltpu.HBM),
        ],
        out_specs=pl.BlockSpec(memory_space=pltpu.HBM),
        out_shape=jax.ShapeDtypeStruct((n,), jnp.float32),
        scratch_shapes=[
            pltpu.VMEM((_NB, k, _BN), jnp.float32),
            pltpu.VMEM((_NB, k, _BN), jnp.float32),
            pltpu.VMEM((_NB, 1, _BN), jnp.float32),
            pltpu.SemaphoreType.DMA((_NB,)),
            pltpu.SemaphoreType.DMA((_NB,)),
            pltpu.SemaphoreType.DMA((_NB,)),
        ],
    )(ut, it)


# speedup vs baseline: 1.0017x; 1.0017x over previous
import jax
import jax.numpy as jnp
from jax import lax
from jax.experimental import pallas as pl
from jax.experimental.pallas import tpu as pltpu

_BN = 16000
_NB = 10


def _body(u_hbm, i_hbm, o_hbm, u_v, i_v, o_v, sem_u, sem_i, sem_o):
    n = u_hbm.shape[1]
    steps = n // _BN

    def issue_in(si, b):
        start = si * _BN
        pltpu.async_copy(u_hbm.at[:, pl.ds(start, _BN)], u_v.at[b],
                         sem_u.at[b])
        pltpu.async_copy(i_hbm.at[:, pl.ds(start, _BN)], i_v.at[b],
                         sem_i.at[b])

    def wait_in(b):
        pltpu.make_async_copy(u_hbm.at[:, pl.ds(0, _BN)], u_v.at[b],
                              sem_u.at[b]).wait()
        pltpu.make_async_copy(i_hbm.at[:, pl.ds(0, _BN)], i_v.at[b],
                              sem_i.at[b]).wait()

    def issue_out(si, b):
        pltpu.async_copy(o_v.at[b, 0], o_hbm.at[pl.ds(si * _BN, _BN)],
                         sem_o.at[b])

    def wait_out(b):
        pltpu.make_async_copy(o_v.at[b, 0], o_hbm.at[pl.ds(0, _BN)],
                              sem_o.at[b]).wait()

    for b in range(_NB - 1):
        issue_in(b, b)

    def grp(g, carry):
        for j in range(_NB):
            si = g * _NB + j
            nxt = si + _NB - 1

            @pl.when(nxt < steps)
            def _():
                issue_in(nxt, (j + _NB - 1) % _NB)

            wait_in(j)

            @pl.when(si >= _NB)
            def _():
                wait_out(j)

            o_v[j, 0, :] = jnp.sum(u_v[j] * i_v[j], axis=0)
            issue_out(si, j)
        return carry

    lax.fori_loop(0, steps // _NB, grp, 0)
    for b in range(_NB):
        wait_out(b)


def kernel(gu, gi):
    gu = jnp.squeeze(gu)
    gi = jnp.squeeze(gi)
    n, k = gu.shape
    ut = gu.T
    it = gi.T
    return pl.pallas_call(
        _body,
        in_specs=[
            pl.BlockSpec(memory_space=pltpu.HBM),
            pl.BlockSpec(memory_space=pltpu.HBM),
        ],
        out_specs=pl.BlockSpec(memory_space=pltpu.HBM),
        out_shape=jax.ShapeDtypeStruct((n,), jnp.float32),
        scratch_shapes=[
            pltpu.VMEM((_NB, k, _BN), jnp.float32),
            pltpu.VMEM((_NB, k, _BN), jnp.float32),
            pltpu.VMEM((_NB, 1, _BN), jnp.float32),
            pltpu.SemaphoreType.DMA((_NB,)),
            pltpu.SemaphoreType.DMA((_NB,)),
            pltpu.SemaphoreType.DMA((_NB,)),
        ],
    )(ut, it)


# speedup vs baseline: 1.0066x; 1.0049x over previous
import jax
import jax.numpy as jnp
from jax.experimental import pallas as pl


def _body(u_ref, i_ref, o_ref):
    o_ref[...] = jnp.sum(u_ref[...] * i_ref[...], axis=0)


def kernel(gu, gi):
    gu = jnp.squeeze(gu)
    gi = jnp.squeeze(gi)
    n, k = gu.shape
    ut = gu.T
    it = gi.T
    bn = 28672
    grid = pl.cdiv(n, bn)
    return pl.pallas_call(
        _body,
        grid=(grid,),
        in_specs=[
            pl.BlockSpec((k, bn), lambda i: (0, i)),
            pl.BlockSpec((k, bn), lambda i: (0, i)),
        ],
        out_specs=pl.BlockSpec((bn,), lambda i: (i,)),
        out_shape=jax.ShapeDtypeStruct((n,), jnp.float32),
    )(ut, it)


# speedup vs baseline: 1.0068x; 1.0002x over previous
import jax
import jax.numpy as jnp
from jax.experimental import pallas as pl


def _body(u_ref, i_ref, o_ref):
    o_ref[...] = jnp.sum(u_ref[...] * i_ref[...], axis=0)


def kernel(gu, gi):
    gu = jnp.squeeze(gu)
    gi = jnp.squeeze(gi)
    n, k = gu.shape
    ut = gu.T
    it = gi.T
    bn = 29696
    grid = pl.cdiv(n, bn)
    return pl.pallas_call(
        _body,
        grid=(grid,),
        in_specs=[
            pl.BlockSpec((k, bn), lambda i: (0, i)),
            pl.BlockSpec((k, bn), lambda i: (0, i)),
        ],
        out_specs=pl.BlockSpec((bn,), lambda i: (i,)),
        out_shape=jax.ShapeDtypeStruct((n,), jnp.float32),
    )(ut, it)
